# baseline (speedup 1.0000x reference)
import functools

import jax
import jax.numpy as jnp
from jax.experimental import pallas as pl
from jax.experimental.pallas import tpu as pltpu


M = 1024
K = 4096
N = 4096
BN = 512
BK = 256
NC = K // BK
NJ = N // BN


def _x_copy(x_hbm, x_vmem, sem, c):
    sl = slice(c * BK, (c + 1) * BK)
    return pltpu.make_async_copy(x_hbm.at[:, sl], x_vmem.at[:, sl], sem.at[c])


def _w_copy(w_hbm, w_vmem, sem, j, slot):
    return pltpu.make_async_copy(
        w_hbm.at[:, pl.ds(j * BN, BN)],
        w_vmem.at[slot],
        sem.at[slot],
    )


def _matmul_kernel(x_hbm, w_hbm, b_ref, out_ref, x_vmem, w_vmem, sem_x, sem_w):
    j = pl.program_id(0)
    slot = jax.lax.rem(j, 2)

    @pl.when(j == 0)
    def _():
        _w_copy(w_hbm, w_vmem, sem_w, 0, 0).start()
        for c in range(NC):
            _x_copy(x_hbm, x_vmem, sem_x, c).start()
        _w_copy(w_hbm, w_vmem, sem_w, 1, 1).start()

    @pl.when((j >= 1) & (j < NJ - 1))
    def _():
        _w_copy(w_hbm, w_vmem, sem_w, j + 1, 1 - slot).start()

    @pl.when(j == 0)
    def _():
        _w_copy(w_hbm, w_vmem, sem_w, j, slot).wait()
        acc = None
        for c in range(NC):
            _x_copy(x_hbm, x_vmem, sem_x, c).wait()
            ksl = slice(c * BK, (c + 1) * BK)
            part = jnp.dot(
                x_vmem[:, ksl], w_vmem[slot, ksl, :],
                preferred_element_type=jnp.float32,
            )
            acc = part if acc is None else acc + part
        out_ref[...] = acc + b_ref[...]

    @pl.when(j > 0)
    def _():
        _w_copy(w_hbm, w_vmem, sem_w, j, slot).wait()
        out_ref[...] = jnp.dot(
            x_vmem[...], w_vmem[slot], preferred_element_type=jnp.float32
        ) + b_ref[...]


@functools.partial(jax.jit)
def kernel(in_values, weight, bias):
    bias2d = bias.reshape(1, N)
    return pl.pallas_call(
        _matmul_kernel,
        grid=(NJ,),
        in_specs=[
            pl.BlockSpec(memory_space=pltpu.MemorySpace.HBM),
            pl.BlockSpec(memory_space=pltpu.MemorySpace.HBM),
            pl.BlockSpec((1, BN), lambda j: (0, j)),
        ],
        out_specs=pl.BlockSpec((M, BN), lambda j: (0, j)),
        out_shape=jax.ShapeDtypeStruct((M, N), jnp.float32),
        scratch_shapes=[
            pltpu.VMEM((M, K), jnp.float32),
            pltpu.VMEM((2, K, BN), jnp.float32),
            pltpu.SemaphoreType.DMA((NC,)),
            pltpu.SemaphoreType.DMA((2,)),
        ],
    )(in_values, weight, bias2d)


# speedup vs baseline: 1.0363x; 1.0363x over previous
import functools

import jax
import jax.numpy as jnp
from jax.experimental import pallas as pl
from jax.experimental.pallas import tpu as pltpu


M = 1024
K = 4096
N = 4096
BN = 512
BK = 512
NC = K // BK
NJ = N // BN


def _x_copy(x_hbm, x_vmem, sem, c):
    sl = slice(c * BK, (c + 1) * BK)
    return pltpu.make_async_copy(x_hbm.at[:, sl], x_vmem.at[:, sl], sem.at[c])


def _w_copy(w_hbm, w_vmem, sem, j, slot):
    return pltpu.make_async_copy(
        w_hbm.at[:, pl.ds(j * BN, BN)],
        w_vmem.at[slot],
        sem.at[slot],
    )


def _matmul_kernel(x_hbm, w_hbm, b_ref, out_ref, x_vmem, w_vmem, sem_x, sem_w):
    j = pl.program_id(0)
    slot = jax.lax.rem(j, 2)

    @pl.when(j == 0)
    def _():
        _w_copy(w_hbm, w_vmem, sem_w, 0, 0).start()
        for c in range(NC):
            _x_copy(x_hbm, x_vmem, sem_x, c).start()
        _w_copy(w_hbm, w_vmem, sem_w, 1, 1).start()

    @pl.when((j >= 1) & (j < NJ - 1))
    def _():
        _w_copy(w_hbm, w_vmem, sem_w, j + 1, 1 - slot).start()

    @pl.when(j == 0)
    def _():
        _w_copy(w_hbm, w_vmem, sem_w, j, slot).wait()
        acc = None
        for c in range(NC):
            _x_copy(x_hbm, x_vmem, sem_x, c).wait()
            ksl = slice(c * BK, (c + 1) * BK)
            part = jnp.dot(
                x_vmem[:, ksl], w_vmem[slot, ksl, :],
                preferred_element_type=jnp.float32,
            )
            acc = part if acc is None else acc + part
        out_ref[...] = acc + b_ref[...]

    @pl.when(j > 0)
    def _():
        _w_copy(w_hbm, w_vmem, sem_w, j, slot).wait()
        out_ref[...] = jnp.dot(
            x_vmem[...], w_vmem[slot], preferred_element_type=jnp.float32
        ) + b_ref[...]


@functools.partial(jax.jit)
def kernel(in_values, weight, bias):
    bias2d = bias.reshape(1, N)
    return pl.pallas_call(
        _matmul_kernel,
        grid=(NJ,),
        in_specs=[
            pl.BlockSpec(memory_space=pltpu.MemorySpace.HBM),
            pl.BlockSpec(memory_space=pltpu.MemorySpace.HBM),
            pl.BlockSpec((1, BN), lambda j: (0, j)),
        ],
        out_specs=pl.BlockSpec((M, BN), lambda j: (0, j)),
        out_shape=jax.ShapeDtypeStruct((M, N), jnp.float32),
        scratch_shapes=[
            pltpu.VMEM((M, K), jnp.float32),
            pltpu.VMEM((2, K, BN), jnp.float32),
            pltpu.SemaphoreType.DMA((NC,)),
            pltpu.SemaphoreType.DMA((2,)),
        ],
    )(in_values, weight, bias2d)
